# baseline (speedup 1.0000x reference)
import math

import jax
import jax.numpy as jnp
from jax.experimental import pallas as pl
from jax.experimental.pallas import tpu as pltpu

D_MODEL = 256
GRID_DIM = 30
HW = GRID_DIM * GRID_DIM
G = 9
GPB = 3


def _body(x_ref, io_ref, pair_ref, o_ref, comb_scr):
    i = pl.program_id(0)

    @pl.when(i == 0)
    def _init():
        dim = D_MODEL // 2
        r = jax.lax.broadcasted_iota(jnp.int32, (HW, 2 * dim), 0)
        c = jax.lax.broadcasted_iota(jnp.int32, (HW, 2 * dim), 1)
        pos = jnp.where(c < dim, r // GRID_DIM, r % GRID_DIM).astype(jnp.float32)
        cl = c % dim
        freq = jnp.exp((cl - cl % 2).astype(jnp.float32) * (-math.log(10000.0) / dim))
        angle = pos * freq
        pos_emb = jnp.where(cl % 2 == 0, jnp.sin(angle), jnp.cos(angle))
        for gg in range(G):
            comb_scr[gg, :, 0:256] = pos_emb
            comb_scr[gg, :, 256:320] = jnp.broadcast_to(
                io_ref[gg % 2, :][None, :], (HW, 64))
            comb_scr[gg, :, 320:384] = jnp.broadcast_to(
                pair_ref[gg // 2, :][None, :], (HW, 64))

    base = (GPB * i) % G
    o_ref[...] = x_ref[...] + comb_scr[pl.ds(base, GPB), :, :]


@jax.jit
def kernel(x, io_table, pair_table):
    B, Gd, H, W, C = x.shape
    xf = x.reshape(B * Gd, H * W, C)
    nblk = (B * Gd) // GPB
    out = pl.pallas_call(
        _body,
        grid=(nblk,),
        in_specs=[
            pl.BlockSpec((GPB, H * W, C), lambda i: (i, 0, 0)),
            pl.BlockSpec(memory_space=pltpu.VMEM),
            pl.BlockSpec(memory_space=pltpu.VMEM),
        ],
        out_specs=pl.BlockSpec((GPB, H * W, C), lambda i: (i, 0, 0)),
        out_shape=jax.ShapeDtypeStruct((B * Gd, H * W, C), x.dtype),
        scratch_shapes=[
            pltpu.VMEM((G, HW, C), jnp.float32),
        ],
    )(xf, io_table, pair_table)
    return out.reshape(B, Gd, H, W, C)


# speedup vs baseline: 1.7930x; 1.7930x over previous
import math

import jax
import jax.numpy as jnp
from jax.experimental import pallas as pl
from jax.experimental.pallas import tpu as pltpu

D_MODEL = 256
GRID_DIM = 30
G = 9
GPB = 3


def _body(x_ref, io_ref, pair_ref, o_ref, comb_scr):
    b = pl.program_id(0)
    j = pl.program_id(1)

    @pl.when(jnp.logical_and(b == 0, j == 0))
    def _init():
        dim = D_MODEL // 2
        h = jax.lax.broadcasted_iota(jnp.int32, (GRID_DIM, GRID_DIM, 2 * dim), 0)
        w = jax.lax.broadcasted_iota(jnp.int32, (GRID_DIM, GRID_DIM, 2 * dim), 1)
        c = jax.lax.broadcasted_iota(jnp.int32, (GRID_DIM, GRID_DIM, 2 * dim), 2)
        pos = jnp.where(c < dim, h, w).astype(jnp.float32)
        cl = c % dim
        freq = jnp.exp((cl - cl % 2).astype(jnp.float32) * (-math.log(10000.0) / dim))
        angle = pos * freq
        pos_emb = jnp.where(cl % 2 == 0, jnp.sin(angle), jnp.cos(angle))
        for gg in range(G):
            comb_scr[gg, :, :, 0:256] = pos_emb
            comb_scr[gg, :, :, 256:320] = jnp.broadcast_to(
                io_ref[gg % 2, :][None, None, :], (GRID_DIM, GRID_DIM, 64))
            comb_scr[gg, :, :, 320:384] = jnp.broadcast_to(
                pair_ref[gg // 2, :][None, None, :], (GRID_DIM, GRID_DIM, 64))

    o_ref[0] = x_ref[0] + comb_scr[pl.ds(GPB * j, GPB), :, :, :]


@jax.jit
def kernel(x, io_table, pair_table):
    B, Gd, H, W, C = x.shape
    return pl.pallas_call(
        _body,
        grid=(B, Gd // GPB),
        in_specs=[
            pl.BlockSpec((1, GPB, H, W, C), lambda b, j: (b, j, 0, 0, 0)),
            pl.BlockSpec(memory_space=pltpu.VMEM),
            pl.BlockSpec(memory_space=pltpu.VMEM),
        ],
        out_specs=pl.BlockSpec((1, GPB, H, W, C), lambda b, j: (b, j, 0, 0, 0)),
        out_shape=jax.ShapeDtypeStruct((B, Gd, H, W, C), x.dtype),
        scratch_shapes=[
            pltpu.VMEM((G, H, W, C), jnp.float32),
        ],
    )(x, io_table, pair_table)
